# baseline (speedup 1.0000x reference)
import functools

import jax
import jax.numpy as jnp
import numpy as np
from jax import lax
from jax.experimental import pallas as pl
from jax.experimental.pallas import tpu as pltpu
from jax.experimental.pallas import tpu_sc as plsc

N = 100000
E = 3200000
IN_DIM = 16
HID = 64
OUT = 32

NC = 2
NS = 16
NPAD = 100096
RN = NPAD // NS
B1 = 200
B2 = 200
EPT1 = E // (NC * NS)
EPT2 = E // NS
U = 5

PR16 = NPAD * 16 // 128
PR1 = NPAD // 128

BLKN = 4352
GRID = NPAD // BLKN
RB = BLKN * 16 // 128


def _fill_rows(buf, nrows, value):
    def body(i, _):
        buf[i, :] = jnp.full((16,), value, jnp.float32)
        return 0
    lax.fori_loop(0, nrows, body, 0)


def _fill_vec(buf, n, value):
    def body(i, _):
        buf[pl.ds(i * 16, 16)] = jnp.full((16,), value, jnp.float32)
        return 0
    lax.fori_loop(0, n // 16, body, 0)
    if n % 16:
        buf[pl.ds(n - 16, 16)] = jnp.full((16,), value, jnp.float32)


def _zero_spmem_stripe(s, bc, accf, zrows, accd=None, zvec=None):
    base = s * RN
    for off in range(0, RN, bc):
        sz = min(bc, RN - off)
        pltpu.sync_copy(zrows.at[pl.ds(0, sz)], accf.at[pl.ds(base + off, sz)])
        if accd is not None:
            pltpu.sync_copy(zvec.at[pl.ds(0, sz)], accd.at[pl.ds(base + off, sz)])


def _sc_edge_pipeline(ebase, nchunks, bc, srch, dsth, table, accf,
                      srcv, dstv, rows, isems, dsems, gsems, ssems,
                      accd=None, ones=None):
    G = nchunks // U

    def issue_src(k, slot):
        pltpu.async_copy(srch.at[pl.ds(ebase + k * bc, bc)], srcv[slot],
                         isems[slot])

    def issue_dst(k, slot):
        pltpu.async_copy(dsth.at[pl.ds(ebase + k * bc, bc)], dstv[slot],
                         dsems[slot])

    def issue_gather(slot):
        pltpu.async_copy(table.at[srcv[slot]], rows[slot], gsems[slot])

    def issue_scatter(slot):
        pltpu.async_copy(rows[slot], accf.at[dstv[slot]], ssems[slot], add=True)
        if accd is not None:
            pltpu.async_copy(ones, accd.at[dstv[slot]], ssems[slot], add=True)

    def wait_scatter(slot):
        pltpu.make_async_copy(rows[slot], accf.at[dstv[slot]],
                              ssems[slot]).wait()
        if accd is not None:
            pltpu.make_async_copy(ones, accd.at[dstv[slot]],
                                  ssems[slot]).wait()

    def wait_src(k, slot):
        pltpu.make_async_copy(srch.at[pl.ds(ebase + k * bc, bc)], srcv[slot],
                              isems[slot]).wait()

    issue_src(0, 0)
    wait_src(0, 0)
    issue_dst(0, 0)
    issue_gather(0)
    issue_src(1, 1)
    wait_src(1, 1)
    issue_dst(1, 1)
    issue_gather(1)
    issue_src(2, 2)

    def outer(g, _):
        for b in range(U):
            i = g * U + b
            o = ebase + i * bc
            jn2 = (b + 2) % U
            jn3 = (b + 3) % U

            def stage_next():
                wait_src(i + 2, jn2)
                if b < 3:
                    @pl.when(g >= 1)
                    def _():
                        wait_scatter(jn2)
                else:
                    wait_scatter(jn2)
                issue_dst(i + 2, jn2)
                issue_gather(jn2)

            if b < 2:
                stage_next()
                issue_src(i + 3, jn3)
            elif b == 2:
                stage_next()

                @pl.when(g < G - 1)
                def _():
                    issue_src(i + 3, jn3)
            else:
                @pl.when(g < G - 1)
                def _():
                    stage_next()
                    issue_src(i + 3, jn3)

            pltpu.make_async_copy(table.at[srcv[b]], rows[b], gsems[b]).wait()
            pltpu.make_async_copy(dsth.at[pl.ds(o, bc)], dstv[b],
                                  dsems[b]).wait()
            issue_scatter(b)
        return 0

    lax.fori_loop(0, G, outer, 0)

    for b in range(U):
        wait_scatter(b)


def _sc_pass1_body(srch, dsth, xp, aggo0, aggo1, dego0, dego1, *scr):
    accf, accd = scr[0], scr[1]
    p = 2
    srcv = list(scr[p:p + U]); p += U
    dstv = list(scr[p:p + U]); p += U
    rows = list(scr[p:p + U]); p += U
    zvec, ones = scr[p], scr[p + 1]; p += 2
    isems = list(scr[p:p + U]); p += U
    dsems = list(scr[p:p + U]); p += U
    gsems = list(scr[p:p + U]); p += U
    ssems = list(scr[p:p + U])

    c = lax.axis_index("c")
    s = lax.axis_index("s")
    _fill_rows(rows[0], B1, 0.0)
    _fill_vec(zvec, B1, 0.0)
    _fill_vec(ones, B1, 1.0)
    _zero_spmem_stripe(s, B1, accf, rows[0], accd, zvec)
    plsc.subcore_barrier()

    ebase = (c * NS + s) * EPT1
    _sc_edge_pipeline(ebase, EPT1 // B1, B1, srch, dsth, xp, accf,
                      srcv, dstv, rows, isems, dsems, gsems, ssems,
                      accd=accd, ones=ones)
    plsc.subcore_barrier()

    def drain(aggo, dego):
        pltpu.sync_copy(accf.at[pl.ds(s * RN, RN)], aggo.at[pl.ds(s * RN, RN)])
        for off in range(0, RN, B1):
            sz = min(B1, RN - off)
            pltpu.sync_copy(accd.at[pl.ds(s * RN + off, sz)],
                            zvec.at[pl.ds(0, sz)])

            def expand(gg, _):
                base = gg * 16
                v = zvec[pl.ds(base, 16)]
                for t in range(16):
                    rows[0][base + t, :] = jnp.full((16,), v[t], jnp.float32)
                return 0
            lax.fori_loop(0, sz // 16, expand, 0)
            if sz % 16:
                base = sz - 16
                v = zvec[pl.ds(base, 16)]
                for t in range(16):
                    rows[0][base + t, :] = jnp.full((16,), v[t], jnp.float32)
            pltpu.sync_copy(rows[0].at[pl.ds(0, sz)],
                            dego.at[pl.ds(s * RN + off, sz)])

    @pl.when(c == 0)
    def _():
        drain(aggo0, dego0)

    @pl.when(c == 1)
    def _():
        drain(aggo1, dego1)


def _sc_pass2_body(srch, dsth, ya, yb, aggo0, aggo1, *scr):
    accf = scr[0]
    p = 1
    srcv = list(scr[p:p + U]); p += U
    dstv = list(scr[p:p + U]); p += U
    rows = list(scr[p:p + U]); p += U
    isems = list(scr[p:p + U]); p += U
    dsems = list(scr[p:p + U]); p += U
    gsems = list(scr[p:p + U]); p += U
    ssems = list(scr[p:p + U])

    c = lax.axis_index("c")
    s = lax.axis_index("s")
    _fill_rows(rows[0], B2, 0.0)
    _zero_spmem_stripe(s, B2, accf, rows[0])
    plsc.subcore_barrier()

    ebase = s * EPT2

    def run(table):
        _sc_edge_pipeline(ebase, EPT2 // B2, B2, srch, dsth, table, accf,
                          srcv, dstv, rows, isems, dsems, gsems, ssems)

    @pl.when(c == 0)
    def _():
        run(ya)

    @pl.when(c == 1)
    def _():
        run(yb)

    plsc.subcore_barrier()

    @pl.when(c == 0)
    def _():
        pltpu.sync_copy(accf.at[pl.ds(s * RN, RN)], aggo0.at[pl.ds(s * RN, RN)])

    @pl.when(c == 1)
    def _():
        pltpu.sync_copy(accf.at[pl.ds(s * RN, RN)], aggo1.at[pl.ds(s * RN, RN)])


@functools.cache
def _sc_passes():
    mesh = plsc.VectorSubcoreMesh(
        core_axis_name="c", subcore_axis_name="s",
        num_cores=NC, num_subcores=NS)
    params = pltpu.CompilerParams(use_tc_tiling_on_sc=False)
    p1 = pl.kernel(
        _sc_pass1_body,
        out_type=[jax.ShapeDtypeStruct((NPAD, IN_DIM), jnp.float32),
                  jax.ShapeDtypeStruct((NPAD, IN_DIM), jnp.float32),
                  jax.ShapeDtypeStruct((NPAD, IN_DIM), jnp.float32),
                  jax.ShapeDtypeStruct((NPAD, IN_DIM), jnp.float32)],
        mesh=mesh,
        scratch_types=(
            [pltpu.VMEM_SHARED((NPAD, IN_DIM), jnp.float32),
             pltpu.VMEM_SHARED((NPAD,), jnp.float32)]
            + [pltpu.VMEM((B1,), jnp.int32)] * (2 * U)
            + [pltpu.VMEM((B1, IN_DIM), jnp.float32)] * U
            + [pltpu.VMEM((B1,), jnp.float32)] * 2
            + [pltpu.SemaphoreType.DMA] * (4 * U)),
        compiler_params=params,
    )
    p2 = pl.kernel(
        _sc_pass2_body,
        out_type=[jax.ShapeDtypeStruct((NPAD, IN_DIM), jnp.float32),
                  jax.ShapeDtypeStruct((NPAD, IN_DIM), jnp.float32)],
        mesh=mesh,
        scratch_types=(
            [pltpu.VMEM_SHARED((NPAD, IN_DIM), jnp.float32)]
            + [pltpu.VMEM((B2,), jnp.int32)] * (2 * U)
            + [pltpu.VMEM((B2, IN_DIM), jnp.float32)] * U
            + [pltpu.SemaphoreType.DMA] * (4 * U)),
        compiler_params=params,
    )
    return p1, p2


def _build_sel():
    sela = np.zeros((256, 128), np.float32)
    selb = np.zeros((256, 128), np.float32)
    s1632 = np.zeros((128, 256), np.float32)
    p256 = np.zeros((256, 256), np.float32)
    for j in range(8):
        for k in range(16):
            sela[32 * j + k, 16 * j + k] = 1.0
            selb[32 * j + 16 + k, 16 * j + k] = 1.0
            s1632[16 * j + k, 32 * j + k] = 1.0
            s1632[16 * j + k, 32 * j + 16 + k] = 1.0
            p256[16 * j + k, 32 * j + k] = 1.0
            p256[128 + 16 * j + k, 32 * j + 16 + k] = 1.0
    m512 = np.kron(np.eye(8, dtype=np.float32),
                   np.full((HID, HID), 1.0 / HID, np.float32))
    m256 = np.kron(np.eye(8, dtype=np.float32),
                   np.full((OUT, OUT), 1.0 / OUT, np.float32))
    return sela, selb, s1632, p256, m512, m256


_SELA, _SELB, _S1632, _P256, _M512, _M256 = _build_sel()


def _dot(a, b):
    return jnp.dot(a, b, preferred_element_type=jnp.float32)


def _ln_seg(v, mseg, g, b, eps=1e-5):
    mu = _dot(v, mseg)
    var = _dot(v * v, mseg) - mu * mu
    return (v - mu) / jnp.sqrt(var + eps) * g + b


def _tc1_body(a0, a1, dg0, dg1, x, w1k, bl1t, wr1k, g1t, b1t, w2k, wr2k,
              sela, selb, s1632, m512, ya, yb, z, r32):
    deg = jnp.maximum(dg0[...] + dg1[...], 1.0)
    recip = 1.0 / deg
    mean = (a0[...] + a1[...]) * recip
    h = _dot(mean, w1k[...]) + bl1t[...] + _dot(x[...], wr1k[...])
    h = jnp.maximum(_ln_seg(h, m512[...], g1t[...], b1t[...]), 0.0)
    y8 = _dot(h, w2k[...])
    ya[...] = _dot(y8, sela[...])
    yb[...] = _dot(y8, selb[...])
    z[...] = _dot(h, wr2k[...])
    r32[...] = _dot(recip, s1632[...])


def _tc2_body(a20, a21, r32, z, p256, m256, bl2t, g2t, b2t, out):
    u = jnp.concatenate([a20[...], a21[...]], axis=1)
    agg = _dot(u, p256[...])
    pre = agg * r32[...] + bl2t[...] + z[...]
    out[...] = jnp.maximum(_ln_seg(pre, m256[...], g2t[...], b2t[...]), 0.0)


def _full(shape):
    return pl.BlockSpec(shape, lambda i: (0,) * len(shape))


def _rows(shape):
    return pl.BlockSpec(shape, lambda i: (i,) + (0,) * (len(shape) - 1))


_tc1 = pl.pallas_call(
    _tc1_body,
    grid=(GRID,),
    in_specs=[
        _rows((RB, 128)), _rows((RB, 128)),
        _rows((RB, 128)), _rows((RB, 128)),
        _rows((RB, 128)),
        _full((128, 8 * HID)), _full((1, 8 * HID)),
        _full((128, 8 * HID)),
        _full((1, 8 * HID)), _full((1, 8 * HID)),
        _full((8 * HID, 8 * OUT)), _full((8 * HID, 8 * OUT)),
        _full((256, 128)), _full((256, 128)), _full((128, 256)),
        _full((8 * HID, 8 * HID)),
    ],
    out_specs=[
        _rows((RB, 128)), _rows((RB, 128)),
        _rows((RB, 256)), _rows((RB, 256)),
    ],
    out_shape=[
        jax.ShapeDtypeStruct((PR16, 128), jnp.float32),
        jax.ShapeDtypeStruct((PR16, 128), jnp.float32),
        jax.ShapeDtypeStruct((PR16, 256), jnp.float32),
        jax.ShapeDtypeStruct((PR16, 256), jnp.float32),
    ],
)

_tc2 = pl.pallas_call(
    _tc2_body,
    grid=(GRID,),
    in_specs=[
        _rows((RB, 128)), _rows((RB, 128)),
        _rows((RB, 256)), _rows((RB, 256)),
        _full((256, 256)),
        _full((8 * OUT, 8 * OUT)),
        _full((1, 8 * OUT)), _full((1, 8 * OUT)), _full((1, 8 * OUT)),
    ],
    out_specs=_rows((RB, 256)),
    out_shape=jax.ShapeDtypeStruct((PR16, 256), jnp.float32),
)


def kernel(edge_index, node_emb, Wl1, bl1, Wr1, g1, b1, Wl2, bl2, Wr2, g2, b2):
    src = edge_index[0]
    dst = edge_index[1]
    f32 = jnp.float32
    eye8 = jnp.eye(8, dtype=f32)

    xp = jnp.pad(node_emb.reshape(N * IN_DIM // 128, 128),
                 ((0, PR16 - N * IN_DIM // 128), (0, 0)))
    xfull = xp.reshape(NPAD, IN_DIM)

    sc_pass1, sc_pass2 = _sc_passes()
    a0, a1, dg0, dg1 = sc_pass1(src, dst, xfull)

    dgx0 = dg0.reshape(PR16, 128)
    dgx1 = dg1.reshape(PR16, 128)

    ya, yb, z, r32 = _tc1(
        a0.reshape(PR16, 128), a1.reshape(PR16, 128), dgx0, dgx1, xp,
        jnp.kron(eye8, Wl1), jnp.tile(bl1, 8).reshape(1, -1),
        jnp.kron(eye8, Wr1),
        jnp.tile(g1, 8).reshape(1, -1), jnp.tile(b1, 8).reshape(1, -1),
        jnp.kron(eye8, Wl2), jnp.kron(eye8, Wr2),
        _SELA, _SELB, _S1632, _M512)

    a20, a21 = sc_pass2(src, dst,
                        ya.reshape(NPAD, IN_DIM), yb.reshape(NPAD, IN_DIM))

    outp = _tc2(a20.reshape(PR16, 128), a21.reshape(PR16, 128), r32, z,
                _P256, _M256,
                jnp.tile(bl2, 8).reshape(1, -1),
                jnp.tile(g2, 8).reshape(1, -1),
                jnp.tile(b2, 8).reshape(1, -1))
    return outp.reshape(NPAD, OUT)[:N]


# speedup vs baseline: 1.2391x; 1.2391x over previous
import functools

import jax
import jax.numpy as jnp
import numpy as np
from jax import lax
from jax.experimental import pallas as pl
from jax.experimental.pallas import tpu as pltpu
from jax.experimental.pallas import tpu_sc as plsc

N = 100000
E = 3200000
IN_DIM = 16
HID = 64
OUT = 32

NC = 2
NS = 16
NPAD = 100096
RN = NPAD // NS
B1 = 200
B2 = 400
EPT1 = E // (NC * NS)
EPT2 = E // NS
U = 4

PR16 = NPAD * 16 // 128
PR1 = NPAD // 128

BLKN = 4352
GRID = NPAD // BLKN
RB = BLKN * 16 // 128


def _fill_rows(buf, nrows, value):
    def body(i, _):
        buf[i, :] = jnp.full((16,), value, jnp.float32)
        return 0
    lax.fori_loop(0, nrows, body, 0)


def _fill_vec(buf, n, value):
    def body(i, _):
        buf[pl.ds(i * 16, 16)] = jnp.full((16,), value, jnp.float32)
        return 0
    lax.fori_loop(0, n // 16, body, 0)
    if n % 16:
        buf[pl.ds(n - 16, 16)] = jnp.full((16,), value, jnp.float32)


def _zero_spmem_stripe(s, bc, accf, zrows, accd=None, zvec=None):
    base = s * RN
    for off in range(0, RN, bc):
        sz = min(bc, RN - off)
        pltpu.sync_copy(zrows.at[pl.ds(0, sz)], accf.at[pl.ds(base + off, sz)])
        if accd is not None:
            pltpu.sync_copy(zvec.at[pl.ds(0, sz)], accd.at[pl.ds(base + off, sz)])


def _sc_edge_pipeline(ebase, nchunks, bc, srch, dsth, table, accf,
                      srcv, dstv, rows, isems, dsems, gsems, ssems,
                      accd=None, ones=None):
    G = nchunks // U

    def issue_src(k, slot):
        pltpu.async_copy(srch.at[pl.ds(ebase + k * bc, bc)], srcv[slot],
                         isems[slot])

    def issue_dst(k, slot):
        pltpu.async_copy(dsth.at[pl.ds(ebase + k * bc, bc)], dstv[slot],
                         dsems[slot])

    def issue_gather(slot):
        pltpu.async_copy(table.at[srcv[slot]], rows[slot], gsems[slot])

    def issue_scatter(slot):
        pltpu.async_copy(rows[slot], accf.at[dstv[slot]], ssems[slot], add=True)
        if accd is not None:
            pltpu.async_copy(ones, accd.at[dstv[slot]], ssems[slot], add=True)

    def wait_scatter(slot):
        pltpu.make_async_copy(rows[slot], accf.at[dstv[slot]],
                              ssems[slot]).wait()
        if accd is not None:
            pltpu.make_async_copy(ones, accd.at[dstv[slot]],
                                  ssems[slot]).wait()

    issue_src(0, 0)
    pltpu.make_async_copy(srch.at[pl.ds(ebase, bc)], srcv[0], isems[0]).wait()
    issue_dst(0, 0)
    issue_gather(0)
    issue_src(1, 1)

    def outer(g, _):
        for b in range(U):
            i = g * U + b
            o = ebase + i * bc
            jn = (b + 1) % U
            jn2 = (b + 2) % U

            def stage_next():
                pltpu.make_async_copy(
                    srch.at[pl.ds(o + bc, bc)], srcv[jn], isems[jn]).wait()
                if b < 3:
                    @pl.when(g >= 1)
                    def _():
                        wait_scatter(jn)
                else:
                    wait_scatter(jn)
                issue_dst(i + 1, jn)
                issue_gather(jn)

            if b < 2:
                stage_next()
                issue_src(i + 2, jn2)
            elif b == 2:
                stage_next()

                @pl.when(g < G - 1)
                def _():
                    issue_src(i + 2, jn2)
            else:
                @pl.when(g < G - 1)
                def _():
                    stage_next()
                    issue_src(i + 2, jn2)

            pltpu.make_async_copy(table.at[srcv[b]], rows[b], gsems[b]).wait()
            pltpu.make_async_copy(dsth.at[pl.ds(o, bc)], dstv[b],
                                  dsems[b]).wait()
            issue_scatter(b)
        return 0

    lax.fori_loop(0, G, outer, 0)

    for b in range(U):
        wait_scatter(b)


def _sc_pass1_body(eidx, xp, aggo0, aggo1, dego0, dego1, *scr):
    srch = eidx.at[0]
    dsth = eidx.at[1]
    accf, accd = scr[0], scr[1]
    srcv, dstv, rows = list(scr[2:6]), list(scr[6:10]), list(scr[10:14])
    zvec, ones = scr[14], scr[15]
    isems, dsems = list(scr[16:20]), list(scr[20:24])
    gsems, ssems = list(scr[24:28]), list(scr[28:32])

    c = lax.axis_index("c")
    s = lax.axis_index("s")
    _fill_rows(rows[0], B1, 0.0)
    _fill_vec(zvec, B1, 0.0)
    _fill_vec(ones, B1, 1.0)
    _zero_spmem_stripe(s, B1, accf, rows[0], accd, zvec)
    plsc.subcore_barrier()

    ebase = (c * NS + s) * EPT1
    _sc_edge_pipeline(ebase, EPT1 // B1, B1, srch, dsth, xp, accf,
                      srcv, dstv, rows, isems, dsems, gsems, ssems,
                      accd=accd, ones=ones)
    plsc.subcore_barrier()

    def drain(aggo, dego):
        pltpu.sync_copy(accf.at[pl.ds(s * RN, RN)], aggo.at[pl.ds(s * RN, RN)])
        for off in range(0, RN, B1):
            sz = min(B1, RN - off)
            pltpu.sync_copy(accd.at[pl.ds(s * RN + off, sz)],
                            zvec.at[pl.ds(0, sz)])

            def expand(gg, _):
                base = gg * 16
                v = zvec[pl.ds(base, 16)]
                for t in range(16):
                    rows[0][base + t, :] = jnp.full((16,), v[t], jnp.float32)
                return 0
            lax.fori_loop(0, sz // 16, expand, 0)
            if sz % 16:
                base = sz - 16
                v = zvec[pl.ds(base, 16)]
                for t in range(16):
                    rows[0][base + t, :] = jnp.full((16,), v[t], jnp.float32)
            pltpu.sync_copy(rows[0].at[pl.ds(0, sz)],
                            dego.at[pl.ds(s * RN + off, sz)])

    @pl.when(c == 0)
    def _():
        drain(aggo0, dego0)

    @pl.when(c == 1)
    def _():
        drain(aggo1, dego1)


def _sc_pass2_body(eidx, ya, yb, aggo0, aggo1, *scr):
    srch = eidx.at[0]
    dsth = eidx.at[1]
    accf = scr[0]
    srcv, dstv, rows = list(scr[1:5]), list(scr[5:9]), list(scr[9:13])
    isems, dsems = list(scr[13:17]), list(scr[17:21])
    gsems, ssems = list(scr[21:25]), list(scr[25:29])

    c = lax.axis_index("c")
    s = lax.axis_index("s")
    _fill_rows(rows[0], B2, 0.0)
    _zero_spmem_stripe(s, B2, accf, rows[0])
    plsc.subcore_barrier()

    ebase = s * EPT2

    def run(table):
        _sc_edge_pipeline(ebase, EPT2 // B2, B2, srch, dsth, table, accf,
                          srcv, dstv, rows, isems, dsems, gsems, ssems)

    @pl.when(c == 0)
    def _():
        run(ya)

    @pl.when(c == 1)
    def _():
        run(yb)

    plsc.subcore_barrier()

    @pl.when(c == 0)
    def _():
        pltpu.sync_copy(accf.at[pl.ds(s * RN, RN)], aggo0.at[pl.ds(s * RN, RN)])

    @pl.when(c == 1)
    def _():
        pltpu.sync_copy(accf.at[pl.ds(s * RN, RN)], aggo1.at[pl.ds(s * RN, RN)])


@functools.cache
def _sc_passes():
    mesh = plsc.VectorSubcoreMesh(
        core_axis_name="c", subcore_axis_name="s",
        num_cores=NC, num_subcores=NS)
    params = pltpu.CompilerParams(use_tc_tiling_on_sc=False)
    p1 = pl.kernel(
        _sc_pass1_body,
        out_type=[jax.ShapeDtypeStruct((NPAD, IN_DIM), jnp.float32),
                  jax.ShapeDtypeStruct((NPAD, IN_DIM), jnp.float32),
                  jax.ShapeDtypeStruct((NPAD, IN_DIM), jnp.float32),
                  jax.ShapeDtypeStruct((NPAD, IN_DIM), jnp.float32)],
        mesh=mesh,
        scratch_types=(
            [pltpu.VMEM_SHARED((NPAD, IN_DIM), jnp.float32),
             pltpu.VMEM_SHARED((NPAD,), jnp.float32)]
            + [pltpu.VMEM((B1,), jnp.int32)] * (2 * U)
            + [pltpu.VMEM((B1, IN_DIM), jnp.float32)] * U
            + [pltpu.VMEM((B1,), jnp.float32)] * 2
            + [pltpu.SemaphoreType.DMA] * (4 * U)),
        compiler_params=params,
    )
    p2 = pl.kernel(
        _sc_pass2_body,
        out_type=[jax.ShapeDtypeStruct((NPAD, IN_DIM), jnp.float32),
                  jax.ShapeDtypeStruct((NPAD, IN_DIM), jnp.float32)],
        mesh=mesh,
        scratch_types=(
            [pltpu.VMEM_SHARED((NPAD, IN_DIM), jnp.float32)]
            + [pltpu.VMEM((B2,), jnp.int32)] * (2 * U)
            + [pltpu.VMEM((B2, IN_DIM), jnp.float32)] * U
            + [pltpu.SemaphoreType.DMA] * (4 * U)),
        compiler_params=params,
    )
    return p1, p2


def _build_sel():
    sela = np.zeros((256, 128), np.float32)
    selb = np.zeros((256, 128), np.float32)
    s1632 = np.zeros((128, 256), np.float32)
    p256 = np.zeros((256, 256), np.float32)
    for j in range(8):
        for k in range(16):
            sela[32 * j + k, 16 * j + k] = 1.0
            selb[32 * j + 16 + k, 16 * j + k] = 1.0
            s1632[16 * j + k, 32 * j + k] = 1.0
            s1632[16 * j + k, 32 * j + 16 + k] = 1.0
            p256[16 * j + k, 32 * j + k] = 1.0
            p256[128 + 16 * j + k, 32 * j + 16 + k] = 1.0
    m512 = np.kron(np.eye(8, dtype=np.float32),
                   np.full((HID, HID), 1.0 / HID, np.float32))
    m256 = np.kron(np.eye(8, dtype=np.float32),
                   np.full((OUT, OUT), 1.0 / OUT, np.float32))
    return sela, selb, s1632, p256, m512, m256


_SELA, _SELB, _S1632, _P256, _M512, _M256 = _build_sel()


def _dot(a, b):
    return jnp.dot(a, b, preferred_element_type=jnp.float32)


def _ln_seg(v, mseg, g, b, eps=1e-5):
    mu = _dot(v, mseg)
    var = _dot(v * v, mseg) - mu * mu
    return (v - mu) / jnp.sqrt(var + eps) * g + b


def _tc1_body(a0, a1, dg0, dg1, x, w1k, bl1t, wr1k, g1t, b1t, w2k, wr2k,
              sela, selb, s1632, m512, ya, yb, z, r32):
    deg = jnp.maximum(dg0[...] + dg1[...], 1.0)
    recip = 1.0 / deg
    mean = (a0[...] + a1[...]) * recip
    h = _dot(mean, w1k[...]) + bl1t[...] + _dot(x[...], wr1k[...])
    h = jnp.maximum(_ln_seg(h, m512[...], g1t[...], b1t[...]), 0.0)
    y8 = _dot(h, w2k[...])
    ya[...] = _dot(y8, sela[...])
    yb[...] = _dot(y8, selb[...])
    z[...] = _dot(h, wr2k[...])
    r32[...] = _dot(recip, s1632[...])


def _tc2_body(a20, a21, r32, z, p256, m256, bl2t, g2t, b2t, out):
    u = jnp.concatenate([a20[...], a21[...]], axis=1)
    agg = _dot(u, p256[...])
    pre = agg * r32[...] + bl2t[...] + z[...]
    out[...] = jnp.maximum(_ln_seg(pre, m256[...], g2t[...], b2t[...]), 0.0)


def _full(shape):
    return pl.BlockSpec(shape, lambda i: (0,) * len(shape))


def _rows(shape):
    return pl.BlockSpec(shape, lambda i: (i,) + (0,) * (len(shape) - 1))


_tc1 = pl.pallas_call(
    _tc1_body,
    grid=(GRID,),
    in_specs=[
        _rows((RB, 128)), _rows((RB, 128)),
        _rows((RB, 128)), _rows((RB, 128)),
        _rows((RB, 128)),
        _full((128, 8 * HID)), _full((1, 8 * HID)),
        _full((128, 8 * HID)),
        _full((1, 8 * HID)), _full((1, 8 * HID)),
        _full((8 * HID, 8 * OUT)), _full((8 * HID, 8 * OUT)),
        _full((256, 128)), _full((256, 128)), _full((128, 256)),
        _full((8 * HID, 8 * HID)),
    ],
    out_specs=[
        _rows((RB, 128)), _rows((RB, 128)),
        _rows((RB, 256)), _rows((RB, 256)),
    ],
    out_shape=[
        jax.ShapeDtypeStruct((PR16, 128), jnp.float32),
        jax.ShapeDtypeStruct((PR16, 128), jnp.float32),
        jax.ShapeDtypeStruct((PR16, 256), jnp.float32),
        jax.ShapeDtypeStruct((PR16, 256), jnp.float32),
    ],
)

_tc2 = pl.pallas_call(
    _tc2_body,
    grid=(GRID,),
    in_specs=[
        _rows((RB, 128)), _rows((RB, 128)),
        _rows((RB, 256)), _rows((RB, 256)),
        _full((256, 256)),
        _full((8 * OUT, 8 * OUT)),
        _full((1, 8 * OUT)), _full((1, 8 * OUT)), _full((1, 8 * OUT)),
    ],
    out_specs=_rows((RB, 256)),
    out_shape=jax.ShapeDtypeStruct((PR16, 256), jnp.float32),
)


def kernel(edge_index, node_emb, Wl1, bl1, Wr1, g1, b1, Wl2, bl2, Wr2, g2, b2):
    f32 = jnp.float32
    eye8 = jnp.eye(8, dtype=f32)

    xp = jnp.pad(node_emb.reshape(N * IN_DIM // 128, 128),
                 ((0, PR16 - N * IN_DIM // 128), (0, 0)))
    xfull = xp.reshape(NPAD, IN_DIM)

    sc_pass1, sc_pass2 = _sc_passes()
    a0, a1, dg0, dg1 = sc_pass1(edge_index, xfull)

    dgx0 = dg0.reshape(PR16, 128)
    dgx1 = dg1.reshape(PR16, 128)

    ya, yb, z, r32 = _tc1(
        a0.reshape(PR16, 128), a1.reshape(PR16, 128), dgx0, dgx1, xp,
        jnp.kron(eye8, Wl1), jnp.tile(bl1, 8).reshape(1, -1),
        jnp.kron(eye8, Wr1),
        jnp.tile(g1, 8).reshape(1, -1), jnp.tile(b1, 8).reshape(1, -1),
        jnp.kron(eye8, Wl2), jnp.kron(eye8, Wr2),
        _SELA, _SELB, _S1632, _M512)

    a20, a21 = sc_pass2(edge_index,
                        ya.reshape(NPAD, IN_DIM), yb.reshape(NPAD, IN_DIM))

    outp = _tc2(a20.reshape(PR16, 128), a21.reshape(PR16, 128), r32, z,
                _P256, _M256,
                jnp.tile(bl2, 8).reshape(1, -1),
                jnp.tile(g2, 8).reshape(1, -1),
                jnp.tile(b2, 8).reshape(1, -1))
    return outp.reshape(NPAD, OUT)[:N]


# speedup vs baseline: 1.2404x; 1.0011x over previous
import functools

import jax
import jax.numpy as jnp
import numpy as np
from jax import lax
from jax.experimental import pallas as pl
from jax.experimental.pallas import tpu as pltpu
from jax.experimental.pallas import tpu_sc as plsc

N = 100000
E = 3200000
IN_DIM = 16
HID = 64
OUT = 32

NC = 2
NS = 16
NPAD = 100096
RN = NPAD // NS
B1 = 200
B2 = 400
EPT1 = E // (NC * NS)
EPT2 = E // NS
U = 4

PR16 = NPAD * 16 // 128

BLKN = 4352
GRID = NPAD // BLKN
RB = BLKN * 16 // 128


def _fill_rows(buf, nrows, value):
    def body(i, _):
        buf[i, :] = jnp.full((16,), value, jnp.float32)
        return 0
    lax.fori_loop(0, nrows, body, 0)


def _fill_vec(buf, n, value):
    def body(i, _):
        buf[pl.ds(i * 16, 16)] = jnp.full((16,), value, jnp.float32)
        return 0
    lax.fori_loop(0, n // 16, body, 0)
    if n % 16:
        buf[pl.ds(n - 16, 16)] = jnp.full((16,), value, jnp.float32)


def _zero_spmem_stripe(s, bc, accf, zrows, accd=None, zvec=None):
    base = s * RN
    for off in range(0, RN, bc):
        sz = min(bc, RN - off)
        pltpu.sync_copy(zrows.at[pl.ds(0, sz)], accf.at[pl.ds(base + off, sz)])
        if accd is not None:
            pltpu.sync_copy(zvec.at[pl.ds(0, sz)], accd.at[pl.ds(base + off, sz)])


def _sc_edge_pipeline(ebase, nchunks, bc, srch, dsth, table, accf,
                      srcv, dstv, rows, isems, dsems, gsems, ssems,
                      accd=None, ones=None):
    G = nchunks // U

    def issue_src(k, slot):
        pltpu.async_copy(srch.at[pl.ds(ebase + k * bc, bc)], srcv[slot],
                         isems[slot])

    def issue_dst(k, slot):
        pltpu.async_copy(dsth.at[pl.ds(ebase + k * bc, bc)], dstv[slot],
                         dsems[slot])

    def issue_gather(slot):
        pltpu.async_copy(table.at[srcv[slot]], rows[slot], gsems[slot])

    def issue_scatter(slot):
        pltpu.async_copy(rows[slot], accf.at[dstv[slot]], ssems[slot], add=True)
        if accd is not None:
            pltpu.async_copy(ones, accd.at[dstv[slot]], ssems[slot], add=True)

    def wait_scatter(slot):
        pltpu.make_async_copy(rows[slot], accf.at[dstv[slot]],
                              ssems[slot]).wait()
        if accd is not None:
            pltpu.make_async_copy(ones, accd.at[dstv[slot]],
                                  ssems[slot]).wait()

    issue_src(0, 0)
    pltpu.make_async_copy(srch.at[pl.ds(ebase, bc)], srcv[0], isems[0]).wait()
    issue_dst(0, 0)
    issue_gather(0)
    issue_src(1, 1)

    def outer(g, _):
        for b in range(U):
            i = g * U + b
            o = ebase + i * bc
            jn = (b + 1) % U
            jn2 = (b + 2) % U

            def stage_next():
                pltpu.make_async_copy(
                    srch.at[pl.ds(o + bc, bc)], srcv[jn], isems[jn]).wait()
                if b < 3:
                    @pl.when(g >= 1)
                    def _():
                        wait_scatter(jn)
                else:
                    wait_scatter(jn)
                issue_dst(i + 1, jn)
                issue_gather(jn)

            if b < 2:
                stage_next()
                issue_src(i + 2, jn2)
            elif b == 2:
                stage_next()

                @pl.when(g < G - 1)
                def _():
                    issue_src(i + 2, jn2)
            else:
                @pl.when(g < G - 1)
                def _():
                    stage_next()
                    issue_src(i + 2, jn2)

            pltpu.make_async_copy(table.at[srcv[b]], rows[b], gsems[b]).wait()
            pltpu.make_async_copy(dsth.at[pl.ds(o, bc)], dstv[b],
                                  dsems[b]).wait()
            issue_scatter(b)
        return 0

    lax.fori_loop(0, G, outer, 0)

    for b in range(U):
        wait_scatter(b)


def _sc_pass1_body(eidx, xp, aggo0, aggo1, dego0, dego1, *scr):
    srch = eidx.at[0]
    dsth = eidx.at[1]
    accf, accd = scr[0], scr[1]
    srcv, dstv, rows = list(scr[2:6]), list(scr[6:10]), list(scr[10:14])
    zvec, ones = scr[14], scr[15]
    isems, dsems = list(scr[16:20]), list(scr[20:24])
    gsems, ssems = list(scr[24:28]), list(scr[28:32])

    c = lax.axis_index("c")
    s = lax.axis_index("s")
    _fill_rows(rows[0], B1, 0.0)
    _fill_vec(zvec, B1, 0.0)
    _fill_vec(ones, B1, 1.0)
    _zero_spmem_stripe(s, B1, accf, rows[0], accd, zvec)
    plsc.subcore_barrier()

    ebase = (c * NS + s) * EPT1
    _sc_edge_pipeline(ebase, EPT1 // B1, B1, srch, dsth, xp, accf,
                      srcv, dstv, rows, isems, dsems, gsems, ssems,
                      accd=accd, ones=ones)
    plsc.subcore_barrier()

    def drain(aggo, dego):
        pltpu.sync_copy(accf.at[pl.ds(s * RN, RN)], aggo.at[pl.ds(s * RN, RN)])
        for off in range(0, RN, B1):
            sz = min(B1, RN - off)
            pltpu.sync_copy(accd.at[pl.ds(s * RN + off, sz)],
                            zvec.at[pl.ds(0, sz)])

            def expand(gg, _):
                base = gg * 16
                v = zvec[pl.ds(base, 16)]
                for t in range(16):
                    rows[0][base + t, :] = jnp.full((16,), v[t], jnp.float32)
                return 0
            lax.fori_loop(0, sz // 16, expand, 0)
            if sz % 16:
                base = sz - 16
                v = zvec[pl.ds(base, 16)]
                for t in range(16):
                    rows[0][base + t, :] = jnp.full((16,), v[t], jnp.float32)
            pltpu.sync_copy(rows[0].at[pl.ds(0, sz)],
                            dego.at[pl.ds(s * RN + off, sz)])

    @pl.when(c == 0)
    def _():
        drain(aggo0, dego0)

    @pl.when(c == 1)
    def _():
        drain(aggo1, dego1)


def _sc_pass2_body(eidx, ya, yb, aggo0, aggo1, *scr):
    srch = eidx.at[0]
    dsth = eidx.at[1]
    accf = scr[0]
    srcv, dstv, rows = list(scr[1:5]), list(scr[5:9]), list(scr[9:13])
    isems, dsems = list(scr[13:17]), list(scr[17:21])
    gsems, ssems = list(scr[21:25]), list(scr[25:29])

    c = lax.axis_index("c")
    s = lax.axis_index("s")
    _fill_rows(rows[0], B2, 0.0)
    _zero_spmem_stripe(s, B2, accf, rows[0])
    plsc.subcore_barrier()

    ebase = s * EPT2

    def run(table):
        _sc_edge_pipeline(ebase, EPT2 // B2, B2, srch, dsth, table, accf,
                          srcv, dstv, rows, isems, dsems, gsems, ssems)

    @pl.when(c == 0)
    def _():
        run(ya)

    @pl.when(c == 1)
    def _():
        run(yb)

    plsc.subcore_barrier()

    @pl.when(c == 0)
    def _():
        pltpu.sync_copy(accf.at[pl.ds(s * RN, RN)], aggo0.at[pl.ds(s * RN, RN)])

    @pl.when(c == 1)
    def _():
        pltpu.sync_copy(accf.at[pl.ds(s * RN, RN)], aggo1.at[pl.ds(s * RN, RN)])


@functools.cache
def _sc_passes():
    mesh = plsc.VectorSubcoreMesh(
        core_axis_name="c", subcore_axis_name="s",
        num_cores=NC, num_subcores=NS)
    params = pltpu.CompilerParams(use_tc_tiling_on_sc=False)
    p1 = pl.kernel(
        _sc_pass1_body,
        out_type=[jax.ShapeDtypeStruct((NPAD, IN_DIM), jnp.float32),
                  jax.ShapeDtypeStruct((NPAD, IN_DIM), jnp.float32),
                  jax.ShapeDtypeStruct((NPAD, IN_DIM), jnp.float32),
                  jax.ShapeDtypeStruct((NPAD, IN_DIM), jnp.float32)],
        mesh=mesh,
        scratch_types=(
            [pltpu.VMEM_SHARED((NPAD, IN_DIM), jnp.float32),
             pltpu.VMEM_SHARED((NPAD,), jnp.float32)]
            + [pltpu.VMEM((B1,), jnp.int32)] * (2 * U)
            + [pltpu.VMEM((B1, IN_DIM), jnp.float32)] * U
            + [pltpu.VMEM((B1,), jnp.float32)] * 2
            + [pltpu.SemaphoreType.DMA] * (4 * U)),
        compiler_params=params,
    )
    p2 = pl.kernel(
        _sc_pass2_body,
        out_type=[jax.ShapeDtypeStruct((NPAD, IN_DIM), jnp.float32),
                  jax.ShapeDtypeStruct((NPAD, IN_DIM), jnp.float32)],
        mesh=mesh,
        scratch_types=(
            [pltpu.VMEM_SHARED((NPAD, IN_DIM), jnp.float32)]
            + [pltpu.VMEM((B2,), jnp.int32)] * (2 * U)
            + [pltpu.VMEM((B2, IN_DIM), jnp.float32)] * U
            + [pltpu.SemaphoreType.DMA] * (4 * U)),
        compiler_params=params,
    )
    return p1, p2


def _build_sel():
    sela = np.zeros((256, 128), np.float32)
    selb = np.zeros((256, 128), np.float32)
    s1632 = np.zeros((128, 256), np.float32)
    p256 = np.zeros((256, 256), np.float32)
    for j in range(8):
        for k in range(16):
            sela[32 * j + k, 16 * j + k] = 1.0
            selb[32 * j + 16 + k, 16 * j + k] = 1.0
            s1632[16 * j + k, 32 * j + k] = 1.0
            s1632[16 * j + k, 32 * j + 16 + k] = 1.0
            p256[16 * j + k, 32 * j + k] = 1.0
            p256[128 + 16 * j + k, 32 * j + 16 + k] = 1.0
    m512 = np.kron(np.eye(8, dtype=np.float32),
                   np.full((HID, HID), 1.0 / HID, np.float32))
    m256 = np.kron(np.eye(8, dtype=np.float32),
                   np.full((OUT, OUT), 1.0 / OUT, np.float32))
    return sela, selb, s1632, p256, m512, m256


_SELA, _SELB, _S1632, _P256, _M512, _M256 = _build_sel()


def _dot(a, b):
    return jnp.dot(a, b, preferred_element_type=jnp.float32)


def _ln_seg(v, mseg, g, b, eps=1e-5):
    mu = _dot(v, mseg)
    var = _dot(v * v, mseg) - mu * mu
    return (v - mu) / jnp.sqrt(var + eps) * g + b


def _tc1_body(a0, a1, dg0, dg1, x, w1k, bl1t, wr1k, g1t, b1t, w2k, wr2k,
              sela, selb, s1632, m512, ya, yb, z, r32):
    deg = jnp.maximum(dg0[...] + dg1[...], 1.0)
    recip = 1.0 / deg
    mean = (a0[...] + a1[...]) * recip
    h = _dot(mean, w1k[...]) + bl1t[...] + _dot(x[...], wr1k[...])
    h = jnp.maximum(_ln_seg(h, m512[...], g1t[...], b1t[...]), 0.0)
    y8 = _dot(h, w2k[...])
    ya[...] = _dot(y8, sela[...])
    yb[...] = _dot(y8, selb[...])
    z[...] = _dot(h, wr2k[...])
    r32[...] = _dot(recip, s1632[...])


def _tc2_body(a20, a21, r32, z, p256, m256, bl2t, g2t, b2t, out):
    u = jnp.concatenate([a20[...], a21[...]], axis=1)
    agg = _dot(u, p256[...])
    pre = agg * r32[...] + bl2t[...] + z[...]
    out[...] = jnp.maximum(_ln_seg(pre, m256[...], g2t[...], b2t[...]), 0.0)


def _full(shape):
    return pl.BlockSpec(shape, lambda i: (0,) * len(shape))


def _rows(shape):
    return pl.BlockSpec(shape, lambda i: (i,) + (0,) * (len(shape) - 1))


_tc1 = pl.pallas_call(
    _tc1_body,
    grid=(GRID,),
    in_specs=[
        _rows((RB, 128)), _rows((RB, 128)),
        _rows((RB, 128)), _rows((RB, 128)),
        _rows((RB, 128)),
        _full((128, 8 * HID)), _full((1, 8 * HID)),
        _full((128, 8 * HID)),
        _full((1, 8 * HID)), _full((1, 8 * HID)),
        _full((8 * HID, 8 * OUT)), _full((8 * HID, 8 * OUT)),
        _full((256, 128)), _full((256, 128)), _full((128, 256)),
        _full((8 * HID, 8 * HID)),
    ],
    out_specs=[
        _rows((RB, 128)), _rows((RB, 128)),
        _rows((RB, 256)), _rows((RB, 256)),
    ],
    out_shape=[
        jax.ShapeDtypeStruct((PR16, 128), jnp.float32),
        jax.ShapeDtypeStruct((PR16, 128), jnp.float32),
        jax.ShapeDtypeStruct((PR16, 256), jnp.float32),
        jax.ShapeDtypeStruct((PR16, 256), jnp.float32),
    ],
)

_tc2 = pl.pallas_call(
    _tc2_body,
    grid=(GRID,),
    in_specs=[
        _rows((RB, 128)), _rows((RB, 128)),
        _rows((RB, 256)), _rows((RB, 256)),
        _full((256, 256)),
        _full((8 * OUT, 8 * OUT)),
        _full((1, 8 * OUT)), _full((1, 8 * OUT)), _full((1, 8 * OUT)),
    ],
    out_specs=_rows((RB, 256)),
    out_shape=jax.ShapeDtypeStruct((PR16, 256), jnp.float32),
)


def kernel(edge_index, node_emb, Wl1, bl1, Wr1, g1, b1, Wl2, bl2, Wr2, g2, b2):
    f32 = jnp.float32
    eye8 = jnp.eye(8, dtype=f32)

    xp = jnp.pad(node_emb.reshape(N * IN_DIM // 128, 128),
                 ((0, PR16 - N * IN_DIM // 128), (0, 0)))
    xfull = xp.reshape(NPAD, IN_DIM)

    sc_pass1, sc_pass2 = _sc_passes()
    a0, a1, dg0, dg1 = sc_pass1(edge_index, xfull)

    dgx0 = dg0.reshape(PR16, 128)
    dgx1 = dg1.reshape(PR16, 128)

    ya, yb, z, r32 = _tc1(
        a0.reshape(PR16, 128), a1.reshape(PR16, 128), dgx0, dgx1, xp,
        jnp.kron(eye8, Wl1), jnp.tile(bl1, 8).reshape(1, -1),
        jnp.kron(eye8, Wr1),
        jnp.tile(g1, 8).reshape(1, -1), jnp.tile(b1, 8).reshape(1, -1),
        jnp.kron(eye8, Wl2), jnp.kron(eye8, Wr2),
        _SELA, _SELB, _S1632, _M512)

    a20, a21 = sc_pass2(edge_index,
                        ya.reshape(NPAD, IN_DIM), yb.reshape(NPAD, IN_DIM))

    outp = _tc2(a20.reshape(PR16, 128), a21.reshape(PR16, 128), r32, z,
                _P256, _M256,
                jnp.tile(bl2, 8).reshape(1, -1),
                jnp.tile(g2, 8).reshape(1, -1),
                jnp.tile(b2, 8).reshape(1, -1))
    return outp.reshape(NPAD, OUT)[:N]
